# baseline (speedup 1.0000x reference)
import jax
import jax.numpy as jnp
from jax import lax
from jax.experimental import pallas as pl

LAMBDA_ = 0.1
KNN = 200
EPS_ = 1e-8
B_ = 1024
D_ = 128
RB = 128
BS3 = 32
MSQ = 7
MAXF_BITS = 0x7F7FFFFF


def _bits_kernel(raw_ref, rawt_ref, bits_ref):
    rb = raw_ref[...]
    rawt = rawt_ref[...]
    sq_rows = jnp.sum(rb * rb, axis=1, keepdims=True)
    sq_all = jnp.sum(rawt * rawt, axis=0, keepdims=True)
    g = jnp.dot(rb, rawt, preferred_element_type=jnp.float32)
    d2 = jnp.maximum(sq_rows + sq_all - 2.0 * g, 0.0)
    bits_ref[...] = lax.bitcast_convert_type(d2, jnp.int32)


def _moments_kernel(bits_ref, z_ref, zt_ref, x_ref, xt_ref,
                    sz_ref, sx_ref, mz_ref, mx_ref):
    i = pl.program_id(0)
    bits = bits_ref[...]

    def body(_, carry):
        lo, hi = carry
        mid = lo + lax.div(hi - lo, 2)
        cnt = jnp.sum((bits <= mid).astype(jnp.int32), axis=1,
                      keepdims=True)
        ge = cnt >= (KNN + 1)
        return jnp.where(ge, lo, mid + 1), jnp.where(ge, mid, hi)

    lo0 = jnp.zeros((RB, 1), jnp.int32)
    hi0 = jnp.full((RB, 1), MAXF_BITS, jnp.int32)
    _, thr = lax.fori_loop(0, 31, body, (lo0, hi0))

    rowid = i * RB + lax.broadcasted_iota(jnp.int32, (RB, B_), 0)
    colid = lax.broadcasted_iota(jnp.int32, (RB, B_), 1)
    w = jnp.logical_and(bits <= thr, rowid != colid)
    wb = w.astype(jnp.bfloat16)
    zb = z_ref[...].astype(jnp.bfloat16)
    ztb = zt_ref[...].astype(jnp.bfloat16)
    xb = x_ref[...].astype(jnp.bfloat16)
    xtb = xt_ref[...].astype(jnp.bfloat16)

    mz_ref[...] = jnp.dot(wb, zb, preferred_element_type=jnp.float32)
    mx_ref[...] = jnp.dot(wb, xb, preferred_element_type=jnp.float32)

    for b in range(RB):
        wrow = wb[b:b + 1, :]
        maskz = ztb * wrow
        sz = jnp.dot(maskz, zb, preferred_element_type=jnp.float32)
        sz_ref[b:b + 1, :, :] = sz.astype(jnp.bfloat16)[None]
        maskx = xtb * wrow
        sx = jnp.dot(maskx, xb, preferred_element_type=jnp.float32)
        sx_ref[b:b + 1, :, :] = sx.astype(jnp.bfloat16)[None]


def _power_kernel(sz_ref, sx_ref, mz_ref, mx_ref, psum_ref):
    j = pl.program_id(0)
    inv_sqrt_k = 1.0 / (KNN ** 0.5)

    def build(s_ref, m_ref, s):
        mu = m_ref[s:s + 1, :] * inv_sqrt_k
        outer = lax.dot_general(mu, mu, (((0,), (0,)), ((), ())),
                                preferred_element_type=jnp.float32)
        return s_ref[s] - outer.astype(jnp.bfloat16)

    bz = [build(sz_ref, mz_ref, s) for s in range(BS3)]
    bx = [build(sx_ref, mx_ref, s) for s in range(BS3)]

    def sq_one(a, step):
        an = jnp.dot(a, a, preferred_element_type=jnp.float32)
        if step == 4:
            an = an * lax.rsqrt(jnp.sum(an * an))
        ab = an.astype(jnp.bfloat16)
        if step == 1:
            ab = ab * jnp.asarray(2.0 ** -36, jnp.bfloat16)
        return ab

    for step in range(MSQ - 1):
        bz = [sq_one(a, step) for a in bz]
        bx = [sq_one(a, step) for a in bx]

    partial = jnp.float32(0.0)
    for s in range(BS3):
        p = jnp.dot(bz[s], bx[s], preferred_element_type=jnp.float32)
        num = jnp.sum(p * p)
        bzf = bz[s].astype(jnp.float32)
        bxf = bx[s].astype(jnp.float32)
        dz = jnp.sum(bzf * bzf)
        dx = jnp.sum(bxf * bxf)
        partial = partial + num / (dz * dx)

    @pl.when(j == 0)
    def _():
        psum_ref[...] = jnp.zeros((1, 1), jnp.float32)

    psum_ref[...] += jnp.full((1, 1), partial, jnp.float32)


@jax.jit
def kernel(latent, raw):
    z = latent.astype(jnp.float32)
    x = raw.astype(jnp.float32)
    zt = z.T
    xt = x.T

    bits = pl.pallas_call(
        _bits_kernel,
        grid=(B_ // RB,),
        in_specs=[
            pl.BlockSpec((RB, D_), lambda i: (i, 0)),
            pl.BlockSpec((D_, B_), lambda i: (0, 0)),
        ],
        out_specs=pl.BlockSpec((RB, B_), lambda i: (i, 0)),
        out_shape=jax.ShapeDtypeStruct((B_, B_), jnp.int32),
    )(x, xt)

    sz, sx, mz, mx = pl.pallas_call(
        _moments_kernel,
        grid=(B_ // RB,),
        in_specs=[
            pl.BlockSpec((RB, B_), lambda i: (i, 0)),
            pl.BlockSpec((B_, D_), lambda i: (0, 0)),
            pl.BlockSpec((D_, B_), lambda i: (0, 0)),
            pl.BlockSpec((B_, D_), lambda i: (0, 0)),
            pl.BlockSpec((D_, B_), lambda i: (0, 0)),
        ],
        out_specs=[
            pl.BlockSpec((RB, D_, D_), lambda i: (i, 0, 0)),
            pl.BlockSpec((RB, D_, D_), lambda i: (i, 0, 0)),
            pl.BlockSpec((RB, D_), lambda i: (i, 0)),
            pl.BlockSpec((RB, D_), lambda i: (i, 0)),
        ],
        out_shape=[
            jax.ShapeDtypeStruct((B_, D_, D_), jnp.bfloat16),
            jax.ShapeDtypeStruct((B_, D_, D_), jnp.bfloat16),
            jax.ShapeDtypeStruct((B_, D_), jnp.float32),
            jax.ShapeDtypeStruct((B_, D_), jnp.float32),
        ],
    )(bits, z, zt, x, xt)

    psum = pl.pallas_call(
        _power_kernel,
        grid=(B_ // BS3,),
        in_specs=[
            pl.BlockSpec((BS3, D_, D_), lambda j: (j, 0, 0)),
            pl.BlockSpec((BS3, D_, D_), lambda j: (j, 0, 0)),
            pl.BlockSpec((BS3, D_), lambda j: (j, 0)),
            pl.BlockSpec((BS3, D_), lambda j: (j, 0)),
        ],
        out_specs=pl.BlockSpec((1, 1), lambda j: (0, 0)),
        out_shape=jax.ShapeDtypeStruct((1, 1), jnp.float32),
    )(sz, sx, mz, mx)

    return (LAMBDA_ * (2.0 - 2.0 * psum[0, 0] / B_)).astype(jnp.float32)


# speedup vs baseline: 1.0291x; 1.0291x over previous
import jax
import jax.numpy as jnp
from jax import lax
from jax.experimental import pallas as pl

LAMBDA_ = 0.1
KNN = 200
EPS_ = 1e-8
B_ = 1024
D_ = 128
RB = 128
BS3 = 64
MSQ = 7
MAXF_BITS = 0x7F7FFFFF


def _bits_kernel(raw_ref, rawt_ref, bits_ref):
    rb = raw_ref[...]
    rawt = rawt_ref[...]
    sq_rows = jnp.sum(rb * rb, axis=1, keepdims=True)
    sq_all = jnp.sum(rawt * rawt, axis=0, keepdims=True)
    g = jnp.dot(rb, rawt, preferred_element_type=jnp.float32)
    d2 = jnp.maximum(sq_rows + sq_all - 2.0 * g, 0.0)
    bits_ref[...] = lax.bitcast_convert_type(d2, jnp.int32)


def _moments_kernel(bits_ref, z_ref, zt_ref, x_ref, xt_ref,
                    sz_ref, sx_ref, mz_ref, mx_ref):
    i = pl.program_id(0)
    bits = bits_ref[...]

    def body(_, carry):
        lo, hi = carry
        mid = lo + lax.div(hi - lo, 2)
        cnt = jnp.sum((bits <= mid).astype(jnp.int32), axis=1,
                      keepdims=True)
        ge = cnt >= (KNN + 1)
        return jnp.where(ge, lo, mid + 1), jnp.where(ge, mid, hi)

    lo0 = jnp.zeros((RB, 1), jnp.int32)
    hi0 = jnp.full((RB, 1), MAXF_BITS, jnp.int32)
    _, thr = lax.fori_loop(0, 31, body, (lo0, hi0))

    rowid = i * RB + lax.broadcasted_iota(jnp.int32, (RB, B_), 0)
    colid = lax.broadcasted_iota(jnp.int32, (RB, B_), 1)
    w = jnp.logical_and(bits <= thr, rowid != colid)
    wb = w.astype(jnp.bfloat16)
    zb = z_ref[...].astype(jnp.bfloat16)
    ztb = zt_ref[...].astype(jnp.bfloat16)
    xb = x_ref[...].astype(jnp.bfloat16)
    xtb = xt_ref[...].astype(jnp.bfloat16)

    mz_ref[...] = jnp.dot(wb, zb, preferred_element_type=jnp.float32)
    mx_ref[...] = jnp.dot(wb, xb, preferred_element_type=jnp.float32)

    for b in range(RB):
        wrow = wb[b:b + 1, :]
        maskz = ztb * wrow
        sz = jnp.dot(maskz, zb, preferred_element_type=jnp.float32)
        sz_ref[b:b + 1, :, :] = sz.astype(jnp.bfloat16)[None]
        maskx = xtb * wrow
        sx = jnp.dot(maskx, xb, preferred_element_type=jnp.float32)
        sx_ref[b:b + 1, :, :] = sx.astype(jnp.bfloat16)[None]


def _power_kernel(sz_ref, sx_ref, mz_ref, mx_ref, psum_ref):
    j = pl.program_id(0)
    inv_sqrt_k = 1.0 / (KNN ** 0.5)

    def build(s_ref, m_ref, s):
        mu = m_ref[s:s + 1, :] * inv_sqrt_k
        outer = lax.dot_general(mu, mu, (((0,), (0,)), ((), ())),
                                preferred_element_type=jnp.float32)
        return s_ref[s] - outer.astype(jnp.bfloat16)

    bz = [build(sz_ref, mz_ref, s) for s in range(BS3)]
    bx = [build(sx_ref, mx_ref, s) for s in range(BS3)]

    def sq_one(a, step):
        an = jnp.dot(a, a, preferred_element_type=jnp.float32)
        if step == 4:
            an = an * lax.rsqrt(jnp.sum(an * an))
        ab = an.astype(jnp.bfloat16)
        if step == 1:
            ab = ab * jnp.asarray(2.0 ** -36, jnp.bfloat16)
        return ab

    for step in range(MSQ - 1):
        bz = [sq_one(a, step) for a in bz]
        bx = [sq_one(a, step) for a in bx]

    partial = jnp.float32(0.0)
    for s in range(BS3):
        p = jnp.dot(bz[s], bx[s], preferred_element_type=jnp.float32)
        num = jnp.sum(p * p)
        bzf = bz[s].astype(jnp.float32)
        bxf = bx[s].astype(jnp.float32)
        dz = jnp.sum(bzf * bzf)
        dx = jnp.sum(bxf * bxf)
        partial = partial + num / (dz * dx)

    @pl.when(j == 0)
    def _():
        psum_ref[...] = jnp.zeros((1, 1), jnp.float32)

    psum_ref[...] += jnp.full((1, 1), partial, jnp.float32)


@jax.jit
def kernel(latent, raw):
    z = latent.astype(jnp.float32)
    x = raw.astype(jnp.float32)
    zt = z.T
    xt = x.T

    bits = pl.pallas_call(
        _bits_kernel,
        grid=(B_ // RB,),
        in_specs=[
            pl.BlockSpec((RB, D_), lambda i: (i, 0)),
            pl.BlockSpec((D_, B_), lambda i: (0, 0)),
        ],
        out_specs=pl.BlockSpec((RB, B_), lambda i: (i, 0)),
        out_shape=jax.ShapeDtypeStruct((B_, B_), jnp.int32),
    )(x, xt)

    sz, sx, mz, mx = pl.pallas_call(
        _moments_kernel,
        grid=(B_ // RB,),
        in_specs=[
            pl.BlockSpec((RB, B_), lambda i: (i, 0)),
            pl.BlockSpec((B_, D_), lambda i: (0, 0)),
            pl.BlockSpec((D_, B_), lambda i: (0, 0)),
            pl.BlockSpec((B_, D_), lambda i: (0, 0)),
            pl.BlockSpec((D_, B_), lambda i: (0, 0)),
        ],
        out_specs=[
            pl.BlockSpec((RB, D_, D_), lambda i: (i, 0, 0)),
            pl.BlockSpec((RB, D_, D_), lambda i: (i, 0, 0)),
            pl.BlockSpec((RB, D_), lambda i: (i, 0)),
            pl.BlockSpec((RB, D_), lambda i: (i, 0)),
        ],
        out_shape=[
            jax.ShapeDtypeStruct((B_, D_, D_), jnp.bfloat16),
            jax.ShapeDtypeStruct((B_, D_, D_), jnp.bfloat16),
            jax.ShapeDtypeStruct((B_, D_), jnp.float32),
            jax.ShapeDtypeStruct((B_, D_), jnp.float32),
        ],
    )(bits, z, zt, x, xt)

    psum = pl.pallas_call(
        _power_kernel,
        grid=(B_ // BS3,),
        in_specs=[
            pl.BlockSpec((BS3, D_, D_), lambda j: (j, 0, 0)),
            pl.BlockSpec((BS3, D_, D_), lambda j: (j, 0, 0)),
            pl.BlockSpec((BS3, D_), lambda j: (j, 0)),
            pl.BlockSpec((BS3, D_), lambda j: (j, 0)),
        ],
        out_specs=pl.BlockSpec((1, 1), lambda j: (0, 0)),
        out_shape=jax.ShapeDtypeStruct((1, 1), jnp.float32),
    )(sz, sx, mz, mx)

    return (LAMBDA_ * (2.0 - 2.0 * psum[0, 0] / B_)).astype(jnp.float32)
